# baseline (speedup 1.0000x reference)
import functools

import jax
import jax.numpy as jnp
from jax import lax
from jax.experimental import pallas as pl
from jax.experimental.pallas import tpu as pltpu
from jax.experimental.pallas import tpu_sc as plsc

N_ROWS = 16384
N_FACT = 32
LANES = 16

_info = plsc.get_sparse_core_info()
_NC, _NS = _info.num_cores, _info.num_subcores
_NW = _NC * _NS
_BPW = N_ROWS // _NW
_CHUNK = 128
_NCH = _BPW // _CHUNK
_IDX_ROWS = N_ROWS // _CHUNK


def _sc_body(x0_hbm, x1_hbm, cust_hbm, prod_hbm, wb_hbm, out_hbm,
             idx_c, idx_p, c_rows, p_rows, wb_v, out_v, sem):
    cid = lax.axis_index("c")
    sid = lax.axis_index("s")
    wid = sid * _NC + cid

    pltpu.sync_copy(x0_hbm.at[pl.ds(wid * _NCH, _NCH)], idx_c)
    pltpu.sync_copy(x1_hbm.at[pl.ds(wid * _NCH, _NCH)], idx_p)
    pltpu.sync_copy(wb_hbm, wb_v)

    cps = []
    for j in range(_NCH):
        cps.append(pltpu.async_copy(
            cust_hbm.at[idx_c.at[j]], c_rows.at[pl.ds(j * _CHUNK, _CHUNK)], sem))
        cps.append(pltpu.async_copy(
            prod_hbm.at[idx_p.at[j]], p_rows.at[pl.ds(j * _CHUNK, _CHUNK)], sem))
    for cp in cps:
        cp.wait()

    w = wb_v[0, :]
    b = wb_v[1, :]

    lanes = lax.iota(jnp.int32, LANES)

    def row_group(g, carry):
        acc = jnp.zeros((LANES,), jnp.float32)
        for u in range(LANES):
            r = g * LANES + u
            c0 = c_rows[r, pl.ds(0, LANES)]
            c1 = c_rows[r, pl.ds(LANES, LANES)]
            p0 = p_rows[r, pl.ds(0, LANES)]
            p1 = p_rows[r, pl.ds(LANES, LANES)]
            s = c0 * p0 + c1 * p1
            acc = jnp.where(lanes == u, jnp.sum(s), acc)
        y = acc * w + b
        out_v[pl.ds(g * LANES, LANES)] = 1.0 / (1.0 + jnp.exp(-y))
        return carry

    lax.fori_loop(0, _BPW // LANES, row_group, 0)
    pltpu.sync_copy(out_v, out_hbm.at[pl.ds(wid * _BPW, _BPW)])


_sc_call = functools.partial(
    pl.kernel,
    mesh=plsc.VectorSubcoreMesh(core_axis_name="c", subcore_axis_name="s"),
    out_type=jax.ShapeDtypeStruct((N_ROWS,), jnp.float32),
    compiler_params=pltpu.CompilerParams(
        needs_layout_passes=False, use_tc_tiling_on_sc=False),
    scratch_types=[
        pltpu.VMEM((_NCH, _CHUNK), jnp.int32),
        pltpu.VMEM((_NCH, _CHUNK), jnp.int32),
        pltpu.VMEM((_BPW, N_FACT), jnp.float32),
        pltpu.VMEM((_BPW, N_FACT), jnp.float32),
        pltpu.VMEM((2, LANES), jnp.float32),
        pltpu.VMEM((_BPW,), jnp.float32),
        pltpu.SemaphoreType.DMA,
    ],
)(_sc_body)


def kernel(x, cust_embedding, prod_embedding, out_w, out_b):
    x0 = x[:, 0].reshape(_IDX_ROWS, _CHUNK)
    x1 = x[:, 1].reshape(_IDX_ROWS, _CHUNK)
    wb = jnp.stack([
        jnp.full((LANES,), out_w[0, 0], jnp.float32),
        jnp.full((LANES,), out_b[0], jnp.float32),
    ])
    out = _sc_call(x0, x1, cust_embedding, prod_embedding, wb)
    return out.reshape(N_ROWS, 1)
